# Initial kernel scaffold; baseline (speedup 1.0000x reference)
import jax
import jax.numpy as jnp
from jax.experimental import pallas as pl


def kernel(item_id_his_batch_ph, time_id_his_batch_ph, cate_his_batch_ph, shop_his_batch_ph, node_his_batch_ph, product_his_batch_ph, brand_his_batch_ph, item_id_batch_ph, time_id_batch_ph, cate_id_batch_ph, shop_id_batch_ph, node_id_batch_ph, product_id_batch_ph, brand_id_batch_ph, item_table, cate_table, shop_table, node_table, product_table, brand_table, time_table):
    raise NotImplementedError("write your pallas kernel here")


# speedup vs baseline: 2.4587x; 2.4587x over previous
import functools

import jax
import jax.numpy as jnp
from jax import lax
from jax.experimental import pallas as pl
from jax.experimental.pallas import tpu as pltpu
from jax.experimental.pallas import tpu_sc as plsc

B = 1024
L = 200
EMB = 32
TEMB = 8
NROWS = B * (L + 1)
OUT_W = 6 * EMB + TEMB
NW = 32
ROWS_PER_W = NROWS // NW
CHUNK = 96
NCHUNK = ROWS_PER_W // CHUNK

_item = (0, EMB)
_cate = (EMB, EMB)
_shop = (2 * EMB, EMB)
_node = (3 * EMB, EMB)
_product = (4 * EMB, EMB)
_brand = (5 * EMB, EMB)
_time = (6 * EMB, TEMB)


def _sc_body(idx_item, idx_cate, idx_shop, idx_node, idx_product, idx_brand,
             idx_time, item_table, cate_table, shop_table, node_table,
             product_table, brand_table, time_table, out,
             idx_v, rows_v, trows_v, sem):
    wid = lax.axis_index("s") * 2 + lax.axis_index("c")
    wbase = wid * ROWS_PER_W

    feats = (
        (idx_item, item_table, _item, rows_v),
        (idx_cate, cate_table, _cate, rows_v),
        (idx_shop, shop_table, _shop, rows_v),
        (idx_node, node_table, _node, rows_v),
        (idx_product, product_table, _product, rows_v),
        (idx_brand, brand_table, _brand, rows_v),
        (idx_time, time_table, _time, trows_v),
    )

    def chunk_body(c, carry):
        base = wbase + c * CHUNK
        for idx_hbm, table, (off, w), buf in feats:
            pltpu.sync_copy(idx_hbm.at[pl.ds(base, CHUNK)], idx_v)
            pltpu.async_copy(table.at[idx_v], buf, sem).wait()
            pltpu.sync_copy(buf, out.at[pl.ds(base, CHUNK), pl.ds(off, w)])
        return carry

    lax.fori_loop(0, NCHUNK, chunk_body, 0)


_mesh = plsc.VectorSubcoreMesh(core_axis_name="c", subcore_axis_name="s")

_gather_all = functools.partial(
    pl.kernel,
    mesh=_mesh,
    compiler_params=pltpu.CompilerParams(use_tc_tiling_on_sc=False),
    out_type=jax.ShapeDtypeStruct((NROWS, OUT_W), jnp.float32),
    scratch_types=[
        pltpu.VMEM((CHUNK,), jnp.int32),
        pltpu.VMEM((CHUNK, EMB), jnp.float32),
        pltpu.VMEM((CHUNK, TEMB), jnp.float32),
        pltpu.SemaphoreType.DMA,
    ],
)(_sc_body)


def kernel(item_id_his_batch_ph, time_id_his_batch_ph, cate_his_batch_ph,
           shop_his_batch_ph, node_his_batch_ph, product_his_batch_ph,
           brand_his_batch_ph, item_id_batch_ph, time_id_batch_ph,
           cate_id_batch_ph, shop_id_batch_ph, node_id_batch_ph,
           product_id_batch_ph, brand_id_batch_ph,
           item_table, cate_table, shop_table, node_table,
           product_table, brand_table, time_table):
    def fuse(tgt, his):
        return jnp.concatenate([tgt[:, None], his], axis=1).reshape(-1)

    out2d = _gather_all(
        fuse(item_id_batch_ph, item_id_his_batch_ph),
        fuse(cate_id_batch_ph, cate_his_batch_ph),
        fuse(shop_id_batch_ph, shop_his_batch_ph),
        fuse(node_id_batch_ph, node_his_batch_ph),
        fuse(product_id_batch_ph, product_his_batch_ph),
        fuse(brand_id_batch_ph, brand_his_batch_ph),
        fuse(time_id_batch_ph, time_id_his_batch_ph),
        item_table, cate_table, shop_table, node_table,
        product_table, brand_table, time_table,
    )
    return out2d.reshape(B, L + 1, OUT_W)


# speedup vs baseline: 3.2466x; 1.3205x over previous
import functools

import jax
import jax.numpy as jnp
from jax import lax
from jax.experimental import pallas as pl
from jax.experimental.pallas import tpu as pltpu
from jax.experimental.pallas import tpu_sc as plsc

B = 1024
L = 200
EMB = 32
TEMB = 8
OUT_W = 6 * EMB + TEMB
NW = 32
B_PER_W = B // NW
BLK = 8
NBLK = B_PER_W // BLK
H0 = 104
H1 = L - H0

OFFS = (0, EMB, 2 * EMB, 3 * EMB, 4 * EMB, 5 * EMB, 6 * EMB)
WIDTHS = (EMB, EMB, EMB, EMB, EMB, EMB, TEMB)


def _sc_body(his_item, his_cate, his_shop, his_node, his_product, his_brand,
             his_time, tgt_item, tgt_cate, tgt_shop, tgt_node, tgt_product,
             tgt_brand, tgt_time, item_table, cate_table, shop_table,
             node_table, product_table, brand_table, time_table, out,
             idx_his_v, tgt_idx_v, tgt_rows_v, tgt_time_v, fb, ft,
             gsem, wsem):
    wid = lax.axis_index("s") * 2 + lax.axis_index("c")
    b0 = wid * B_PER_W

    his = (his_item, his_cate, his_shop, his_node, his_product, his_brand,
           his_time)
    tgt = (tgt_item, tgt_cate, tgt_shop, tgt_node, tgt_product, tgt_brand,
           tgt_time)
    tables = (item_table, cate_table, shop_table, node_table, product_table,
              brand_table, time_table)

    for f in range(7):
        pltpu.sync_copy(tgt[f].at[pl.ds(b0, B_PER_W)], tgt_idx_v.at[f])
    gcp = [pltpu.async_copy(tables[f].at[tgt_idx_v.at[f]], tgt_rows_v.at[f],
                            gsem) for f in range(6)]
    gcp.append(pltpu.async_copy(tables[6].at[tgt_idx_v.at[6]], tgt_time_v,
                                gsem))
    for c in gcp:
        c.wait()
    for f in range(6):
        pltpu.sync_copy(tgt_rows_v.at[f],
                        out.at[pl.ds(b0, B_PER_W), 0, pl.ds(OFFS[f], EMB)])
    pltpu.sync_copy(tgt_time_v,
                    out.at[pl.ds(b0, B_PER_W), 0, pl.ds(OFFS[6], TEMB)])

    def blk_body(blk, carry):
        bb = b0 + blk * BLK
        for f in range(7):
            pltpu.sync_copy(his[f].at[pl.ds(bb, BLK)], idx_his_v.at[f])
        wb = [None, None]
        for j in range(BLK):
            p = j % 2
            if wb[p] is not None:
                for c in wb[p]:
                    c.wait()
            gc = []
            for f in range(7):
                buf = fb.at[p, f] if f < 6 else ft.at[p]
                gc.append(pltpu.async_copy(
                    tables[f].at[idx_his_v.at[f, j, pl.ds(0, H0)]],
                    buf.at[pl.ds(0, H0)], gsem))
                gc.append(pltpu.async_copy(
                    tables[f].at[idx_his_v.at[f, j, pl.ds(H0, H1)]],
                    buf.at[pl.ds(H0, H1)], gsem))
            for c in gc:
                c.wait()
            wb[p] = [pltpu.async_copy(
                fb.at[p, f],
                out.at[bb + j, pl.ds(1, L), pl.ds(OFFS[f], EMB)], wsem)
                for f in range(6)]
            wb[p].append(pltpu.async_copy(
                ft.at[p], out.at[bb + j, pl.ds(1, L), pl.ds(OFFS[6], TEMB)],
                wsem))
        for cs in wb:
            if cs is not None:
                for c in cs:
                    c.wait()
        return carry

    lax.fori_loop(0, NBLK, blk_body, 0)


_mesh = plsc.VectorSubcoreMesh(core_axis_name="c", subcore_axis_name="s")

_gather_all = functools.partial(
    pl.kernel,
    mesh=_mesh,
    compiler_params=pltpu.CompilerParams(use_tc_tiling_on_sc=False),
    out_type=jax.ShapeDtypeStruct((B, L + 1, OUT_W), jnp.float32),
    scratch_types=[
        pltpu.VMEM((7, BLK, L), jnp.int32),
        pltpu.VMEM((7, B_PER_W), jnp.int32),
        pltpu.VMEM((6, B_PER_W, EMB), jnp.float32),
        pltpu.VMEM((B_PER_W, TEMB), jnp.float32),
        pltpu.VMEM((2, 6, L, EMB), jnp.float32),
        pltpu.VMEM((2, L, TEMB), jnp.float32),
        pltpu.SemaphoreType.DMA,
        pltpu.SemaphoreType.DMA,
    ],
)(_sc_body)


def kernel(item_id_his_batch_ph, time_id_his_batch_ph, cate_his_batch_ph,
           shop_his_batch_ph, node_his_batch_ph, product_his_batch_ph,
           brand_his_batch_ph, item_id_batch_ph, time_id_batch_ph,
           cate_id_batch_ph, shop_id_batch_ph, node_id_batch_ph,
           product_id_batch_ph, brand_id_batch_ph,
           item_table, cate_table, shop_table, node_table,
           product_table, brand_table, time_table):
    return _gather_all(
        item_id_his_batch_ph, cate_his_batch_ph, shop_his_batch_ph,
        node_his_batch_ph, product_his_batch_ph, brand_his_batch_ph,
        time_id_his_batch_ph,
        item_id_batch_ph, cate_id_batch_ph, shop_id_batch_ph,
        node_id_batch_ph, product_id_batch_ph, brand_id_batch_ph,
        time_id_batch_ph,
        item_table, cate_table, shop_table, node_table,
        product_table, brand_table, time_table,
    )


# speedup vs baseline: 4.4839x; 1.3811x over previous
import functools

import jax
import jax.numpy as jnp
from jax import lax
from jax.experimental import pallas as pl
from jax.experimental.pallas import tpu as pltpu
from jax.experimental.pallas import tpu_sc as plsc

B = 1024
L = 200
EMB = 32
TEMB = 8
OUT_W = 6 * EMB + TEMB
LP = 208
NW = 32
B_PER_W = B // NW
BLK = 8
NBLK = B_PER_W // BLK
H0 = 104
H1 = L - H0
G = 8

PLANE = (0, 0, 0, 0, 1, 1, 1)
OFFS = (0, EMB, 2 * EMB, 3 * EMB, 0, EMB, 2 * EMB)
WIDTHS = (EMB, EMB, EMB, EMB, EMB, EMB, TEMB)
CVT_OFFS = tuple(range(0, L - 16, 16)) + (L - 16,)


def _sc_body(his_item, his_cate, his_shop, his_node, his_product, his_brand,
             his_time, tgt_item, tgt_cate, tgt_shop, tgt_node, tgt_product,
             tgt_brand, tgt_time, item_table, cate_table, shop_table,
             node_table, product_table, brand_table, time_table, out_a, out_b,
             idx_f32_v, idx_i32_v, tgt_idx_v, tgt_rows_v, tgt_time_v, fb, ft,
             gsem, wsem):
    wid = lax.axis_index("s") * 2 + lax.axis_index("c")
    b0 = wid * B_PER_W

    his = (his_item, his_cate, his_shop, his_node, his_product, his_brand,
           his_time)
    tgt = (tgt_item, tgt_cate, tgt_shop, tgt_node, tgt_product, tgt_brand,
           tgt_time)
    tables = (item_table, cate_table, shop_table, node_table, product_table,
              brand_table, time_table)
    planes = (out_a, out_a, out_a, out_a, out_b, out_b, out_b)

    for f in range(7):
        pltpu.sync_copy(tgt[f].at[pl.ds(b0, B_PER_W)], tgt_idx_v.at[f])
    gcp = [pltpu.async_copy(tables[f].at[tgt_idx_v.at[f]], tgt_rows_v.at[f],
                            gsem) for f in range(6)]
    gcp.append(pltpu.async_copy(tables[6].at[tgt_idx_v.at[6]], tgt_time_v,
                                gsem))
    for c in gcp:
        c.wait()
    for f in range(6):
        pltpu.sync_copy(
            tgt_rows_v.at[f],
            planes[f].at[pl.ds(b0, B_PER_W), 0, pl.ds(OFFS[f], EMB)])
    pltpu.sync_copy(
        tgt_time_v, out_b.at[pl.ds(b0, B_PER_W), 0, pl.ds(OFFS[6], TEMB)])

    def blk_body(blk, carry):
        bb = b0 + blk * BLK
        for f in range(7):
            pltpu.sync_copy(his[f].at[pl.ds(bb, BLK)], idx_f32_v.at[f])

        def cvt_row(r, c2):
            for f in range(7):
                for c in CVT_OFFS:
                    idx_i32_v[f, r, pl.ds(c, 16)] = (
                        idx_f32_v[f, r, pl.ds(c, 16)].astype(jnp.int32))
            return c2

        lax.fori_loop(0, BLK, cvt_row, 0)

        wb = [None, None]
        for j in range(BLK):
            p = j % 2
            if wb[p] is not None:
                for c in wb[p]:
                    c.wait()
            gc = []
            for f in range(7):
                buf = fb.at[p, f] if f < 6 else ft.at[p]
                gc.append(pltpu.async_copy(
                    tables[f].at[idx_i32_v.at[f, j, pl.ds(0, H0)]],
                    buf.at[pl.ds(0, H0)], gsem))
                gc.append(pltpu.async_copy(
                    tables[f].at[idx_i32_v.at[f, j, pl.ds(H0, H1)]],
                    buf.at[pl.ds(H0, H1)], gsem))
            for c in gc:
                c.wait()
            wb[p] = [pltpu.async_copy(
                fb.at[p, f],
                planes[f].at[bb + j, pl.ds(1, L), pl.ds(OFFS[f], EMB)], wsem)
                for f in range(6)]
            wb[p].append(pltpu.async_copy(
                ft.at[p], out_b.at[bb + j, pl.ds(1, L), pl.ds(OFFS[6], TEMB)],
                wsem))
        for cs in wb:
            if cs is not None:
                for c in cs:
                    c.wait()
        return carry

    lax.fori_loop(0, NBLK, blk_body, 0)


_mesh = plsc.VectorSubcoreMesh(core_axis_name="c", subcore_axis_name="s")

_gather_planes = functools.partial(
    pl.kernel,
    mesh=_mesh,
    compiler_params=pltpu.CompilerParams(use_tc_tiling_on_sc=False),
    out_type=(jax.ShapeDtypeStruct((B, LP, 128), jnp.float32),
              jax.ShapeDtypeStruct((B, LP, 128), jnp.float32)),
    scratch_types=[
        pltpu.VMEM((7, BLK, L), jnp.float32),
        pltpu.VMEM((7, BLK, L), jnp.int32),
        pltpu.VMEM((7, B_PER_W), jnp.int32),
        pltpu.VMEM((6, B_PER_W, EMB), jnp.float32),
        pltpu.VMEM((B_PER_W, TEMB), jnp.float32),
        pltpu.VMEM((2, 6, L, EMB), jnp.float32),
        pltpu.VMEM((2, L, TEMB), jnp.float32),
        pltpu.SemaphoreType.DMA,
        pltpu.SemaphoreType.DMA,
    ],
)(_sc_body)


def _tc_body(a_ref, b_ref, o_ref):
    a = a_ref[...].reshape(G, LP, 128)
    b = b_ref[...].reshape(G, LP, 128)
    o_ref[:, :, 0:128] = a[:, :L + 1, :]
    o_ref[:, :, 128:OUT_W] = b[:, :L + 1, :OUT_W - 128]


_assemble = pl.pallas_call(
    _tc_body,
    grid=(B // G,),
    in_specs=[pl.BlockSpec((G * LP * 128,), lambda g: (g,)),
              pl.BlockSpec((G * LP * 128,), lambda g: (g,))],
    out_specs=pl.BlockSpec((G, L + 1, OUT_W), lambda g: (g, 0, 0)),
    out_shape=jax.ShapeDtypeStruct((B, L + 1, OUT_W), jnp.float32),
)


def kernel(item_id_his_batch_ph, time_id_his_batch_ph, cate_his_batch_ph,
           shop_his_batch_ph, node_his_batch_ph, product_his_batch_ph,
           brand_his_batch_ph, item_id_batch_ph, time_id_batch_ph,
           cate_id_batch_ph, shop_id_batch_ph, node_id_batch_ph,
           product_id_batch_ph, brand_id_batch_ph,
           item_table, cate_table, shop_table, node_table,
           product_table, brand_table, time_table):
    f32 = jnp.float32
    plane_a, plane_b = _gather_planes(
        item_id_his_batch_ph.astype(f32), cate_his_batch_ph.astype(f32),
        shop_his_batch_ph.astype(f32), node_his_batch_ph.astype(f32),
        product_his_batch_ph.astype(f32), brand_his_batch_ph.astype(f32),
        time_id_his_batch_ph.astype(f32),
        item_id_batch_ph, cate_id_batch_ph, shop_id_batch_ph,
        node_id_batch_ph, product_id_batch_ph, brand_id_batch_ph,
        time_id_batch_ph,
        item_table, cate_table, shop_table, node_table,
        product_table, brand_table, time_table,
    )
    return _assemble(plane_a.reshape(-1), plane_b.reshape(-1))


# speedup vs baseline: 4.7783x; 1.0656x over previous
import functools

import jax
import jax.numpy as jnp
from jax import lax
from jax.experimental import pallas as pl
from jax.experimental.pallas import tpu as pltpu
from jax.experimental.pallas import tpu_sc as plsc

B = 1024
L = 200
EMB = 32
TEMB = 8
OUT_W = 6 * EMB + TEMB
LP = 208
NW = 32
B_PER_W = B // NW
BLK = 8
NBLK = B_PER_W // BLK
H0 = 104
H1 = L - H0
G = 8

PLANE = (0, 0, 0, 0, 1, 1, 1)
OFFS = (0, EMB, 2 * EMB, 3 * EMB, 0, EMB, 2 * EMB)
WIDTHS = (EMB, EMB, EMB, EMB, EMB, EMB, TEMB)
CVT_OFFS = tuple(range(0, L - 16, 16)) + (L - 16,)


def _sc_body(his_item, his_cate, his_shop, his_node, his_product, his_brand,
             his_time, tgt_item, tgt_cate, tgt_shop, tgt_node, tgt_product,
             tgt_brand, tgt_time, item_table, cate_table, shop_table,
             node_table, product_table, brand_table, time_table, out_a, out_b,
             idx_f32_v, idx_i32_v, tgt_idx_v, tgt_rows_v, tgt_time_v, fb, ft,
             gsem, wsem):
    wid = lax.axis_index("s") * 2 + lax.axis_index("c")
    b0 = wid * B_PER_W

    his = (his_item, his_cate, his_shop, his_node, his_product, his_brand,
           his_time)
    tgt = (tgt_item, tgt_cate, tgt_shop, tgt_node, tgt_product, tgt_brand,
           tgt_time)
    tables = (item_table, cate_table, shop_table, node_table, product_table,
              brand_table, time_table)
    planes = (out_a, out_a, out_a, out_a, out_b, out_b, out_b)

    for f in range(7):
        pltpu.sync_copy(tgt[f].at[pl.ds(b0, B_PER_W)], tgt_idx_v.at[f])
    gcp = [pltpu.async_copy(tables[f].at[tgt_idx_v.at[f]], tgt_rows_v.at[f],
                            gsem) for f in range(6)]
    gcp.append(pltpu.async_copy(tables[6].at[tgt_idx_v.at[6]], tgt_time_v,
                                gsem))
    for c in gcp:
        c.wait()
    for f in range(6):
        pltpu.sync_copy(
            tgt_rows_v.at[f],
            planes[f].at[pl.ds(b0, B_PER_W), 0, pl.ds(OFFS[f], EMB)])
    pltpu.sync_copy(
        tgt_time_v, out_b.at[pl.ds(b0, B_PER_W), 0, pl.ds(OFFS[6], TEMB)])

    def blk_body(blk, carry):
        bb = b0 + blk * BLK
        for f in range(7):
            pltpu.sync_copy(his[f].at[pl.ds(bb, BLK)], idx_f32_v.at[f])

        def cvt_row(r, c2):
            for f in range(7):
                for c in CVT_OFFS:
                    idx_i32_v[f, r, pl.ds(c, 16)] = (
                        idx_f32_v[f, r, pl.ds(c, 16)].astype(jnp.int32))
            return c2

        lax.fori_loop(0, BLK, cvt_row, 0)

        wb = [None, None]
        for j in range(BLK):
            p = j % 2
            if wb[p] is not None:
                for c in wb[p]:
                    c.wait()
            gc = []
            for f in range(7):
                buf = fb.at[p, f] if f < 6 else ft.at[p]
                gc.append(pltpu.async_copy(
                    tables[f].at[idx_i32_v.at[f, j, pl.ds(0, H0)]],
                    buf.at[pl.ds(0, H0)], gsem))
                gc.append(pltpu.async_copy(
                    tables[f].at[idx_i32_v.at[f, j, pl.ds(H0, H1)]],
                    buf.at[pl.ds(H0, H1)], gsem))
            for c in gc:
                c.wait()
            wb[p] = [pltpu.async_copy(
                fb.at[p, f],
                planes[f].at[bb + j, pl.ds(1, L), pl.ds(OFFS[f], EMB)], wsem)
                for f in range(6)]
            wb[p].append(pltpu.async_copy(
                ft.at[p], out_b.at[bb + j, pl.ds(1, L), pl.ds(OFFS[6], TEMB)],
                wsem))
        for cs in wb:
            if cs is not None:
                for c in cs:
                    c.wait()
        return carry

    lax.fori_loop(0, NBLK, blk_body, 0)


_mesh = plsc.VectorSubcoreMesh(core_axis_name="c", subcore_axis_name="s")

_gather_planes = functools.partial(
    pl.kernel,
    mesh=_mesh,
    compiler_params=pltpu.CompilerParams(use_tc_tiling_on_sc=False),
    out_type=(jax.ShapeDtypeStruct((B, LP, 128), jnp.float32),
              jax.ShapeDtypeStruct((B, LP, 128), jnp.float32)),
    scratch_types=[
        pltpu.VMEM((7, BLK, L), jnp.float32),
        pltpu.VMEM((7, BLK, L), jnp.int32),
        pltpu.VMEM((7, B_PER_W), jnp.int32),
        pltpu.VMEM((6, B_PER_W, EMB), jnp.float32),
        pltpu.VMEM((B_PER_W, TEMB), jnp.float32),
        pltpu.VMEM((2, 6, L, EMB), jnp.float32),
        pltpu.VMEM((2, L, TEMB), jnp.float32),
        pltpu.SemaphoreType.DMA,
        pltpu.SemaphoreType.DMA,
    ],
)(_sc_body)


def _tc_body(a_ref, b_ref, o_ref):
    a = a_ref[...].reshape(G, LP, 128)
    b = b_ref[...].reshape(G, LP, 128)
    o_ref[:, :, 0:128] = a[:, :L + 1, :]
    o_ref[:, :, 128:OUT_W] = b[:, :L + 1, :OUT_W - 128]


_assemble = pl.pallas_call(
    _tc_body,
    grid=(B // G,),
    in_specs=[pl.BlockSpec((G * LP * 128,), lambda g: (g,)),
              pl.BlockSpec((G * LP * 128,), lambda g: (g,))],
    out_specs=pl.BlockSpec((G, L + 1, OUT_W), lambda g: (g, 0, 0)),
    out_shape=jax.ShapeDtypeStruct((B, L + 1, OUT_W), jnp.float32),
)


def kernel(item_id_his_batch_ph, time_id_his_batch_ph, cate_his_batch_ph,
           shop_his_batch_ph, node_his_batch_ph, product_his_batch_ph,
           brand_his_batch_ph, item_id_batch_ph, time_id_batch_ph,
           cate_id_batch_ph, shop_id_batch_ph, node_id_batch_ph,
           product_id_batch_ph, brand_id_batch_ph,
           item_table, cate_table, shop_table, node_table,
           product_table, brand_table, time_table):
    f32 = jnp.float32
    plane_a, plane_b = _gather_planes(
        item_id_his_batch_ph.astype(f32), cate_his_batch_ph.astype(f32),
        shop_his_batch_ph.astype(f32), node_his_batch_ph.astype(f32),
        product_his_batch_ph.astype(f32), brand_his_batch_ph.astype(f32),
        time_id_his_batch_ph.astype(f32),
        item_id_batch_ph, cate_id_batch_ph, shop_id_batch_ph,
        node_id_batch_ph, product_id_batch_ph, brand_id_batch_ph,
        time_id_batch_ph,
        item_table, cate_table, shop_table, node_table,
        product_table, brand_table, time_table,
    )
    return jnp.concatenate(
        [plane_a[:, :L + 1, :], plane_b[:, :L + 1, :OUT_W - 128]], axis=2)
